# Initial kernel scaffold; baseline (speedup 1.0000x reference)
import jax
import jax.numpy as jnp
from jax.experimental import pallas as pl


def kernel(x_user, x_item, edge_index_ui, edge_index_iu, conv1_ui_Wl, conv1_ui_bl, conv1_ui_Wr, conv1_iu_Wl, conv1_iu_bl, conv1_iu_Wr, lin1_user_W, lin1_user_b, lin1_item_W, lin1_item_b, conv2_ui_Wl, conv2_ui_bl, conv2_ui_Wr, conv2_iu_Wl, conv2_iu_bl, conv2_iu_Wr, lin2_user_W, lin2_user_b, lin2_item_W, lin2_item_b):
    raise NotImplementedError("write your pallas kernel here")


# speedup vs baseline: 1.4238x; 1.4238x over previous
import jax
import jax.numpy as jnp
from jax import lax
from jax.experimental import pallas as pl
from jax.experimental.pallas import tpu as pltpu
from jax.experimental.pallas import tpu_sc as plsc

D = 128
CW = 32
NCORES = 2
NTILES = 16
B = 128


def _mesh():
    return plsc.VectorSubcoreMesh(
        core_axis_name="c", subcore_axis_name="s",
        num_cores=NCORES, num_subcores=NTILES)


def _params():
    return pltpu.CompilerParams(use_tc_tiling_on_sc=False)


def _fill(buf, val):
    v = jnp.full((16,), val, jnp.float32)

    def row(i, carry):
        buf[i, pl.ds(0, 16)] = v
        buf[i, pl.ds(16, 16)] = v
        return carry

    lax.fori_loop(0, buf.shape[0], row, 0)


def _localize(idst, half, trash):
    nb = idst.shape[0]
    base = lax.axis_index("c") * half
    tr = jnp.full((16,), trash, jnp.int32)

    def row(b, carry):
        for v in range(B // 16):
            x = idst[b, pl.ds(v * 16, 16)]
            l = x - jnp.full((16,), base, jnp.int32)
            keep = (l >= 0) & (l < half)
            idst[b, pl.ds(v * 16, 16)] = jnp.where(keep, l, tr)
        return carry

    lax.fori_loop(0, nb, row, 0)


def _zero_acc(zbuf, acc, sid, zrows):
    def zr(j, carry):
        pltpu.sync_copy(zbuf, acc.at[pl.ds(sid * zrows + j * B, B)])
        return carry
    lax.fori_loop(0, zrows // B, zr, 0)


def _sc_seg_sum(ys, src3, dst3, half):
    nb = src3.shape[1]
    accr = half + NTILES * B
    trash = half + NTILES * B // 2
    fl = half // NTILES
    zr = accr // NTILES

    def body(y0, y1, y2, y3, srcr, dstr, o0, o1, o2, o3,
             isrc, idst, rows, zbuf, acc, sem):
        cid = lax.axis_index("c")
        sid = lax.axis_index("s")
        _fill(zbuf, 0.0)
        pltpu.sync_copy(srcr.at[sid], isrc)
        pltpu.sync_copy(dstr.at[sid], idst)
        _localize(idst, half, trash)
        yrefs = (y0, y1, y2, y3)
        orefs = (o0, o1, o2, o3)
        for p in range(4):
            _zero_acc(zbuf, acc, sid, zr)
            plsc.subcore_barrier()

            def bb(b, carry):
                pltpu.async_copy(yrefs[p].at[isrc.at[b]], rows, sem).wait()
                pltpu.sync_copy(rows, acc.at[idst.at[b]], add=True)
                return carry
            lax.fori_loop(0, nb, bb, 0)
            plsc.subcore_barrier()
            pltpu.sync_copy(
                acc.at[pl.ds(sid * fl, fl)],
                orefs[p].at[pl.ds(cid * half + sid * fl, fl)])
            plsc.subcore_barrier()

    out_t = tuple(jax.ShapeDtypeStruct((2 * half, CW), jnp.float32)
                  for _ in range(4))
    scratch = [
        pltpu.VMEM((nb, B), jnp.int32),
        pltpu.VMEM((nb, B), jnp.int32),
        pltpu.VMEM((B, CW), jnp.float32),
        pltpu.VMEM((B, CW), jnp.float32),
        pltpu.VMEM_SHARED((accr, CW), jnp.float32),
        pltpu.SemaphoreType.DMA,
    ]
    return pl.kernel(body, out_type=out_t, mesh=_mesh(),
                     scratch_types=scratch,
                     compiler_params=_params())(*ys, src3, dst3)


def _sc_counts(dst3_ui, dst3_iu, half):
    nb = dst3_ui.shape[1]
    accr = half + NTILES * B
    trash = half + NTILES * B // 2
    fl = half // NTILES
    zr = accr // NTILES

    def body(dui, diu, o_ui, o_iu, id_ui, id_iu, onesb, zbuf, acc):
        cid = lax.axis_index("c")
        sid = lax.axis_index("s")
        _fill(onesb, 1.0)
        _fill(zbuf, 0.0)
        pltpu.sync_copy(dui.at[sid], id_ui)
        pltpu.sync_copy(diu.at[sid], id_iu)
        _localize(id_ui, half, trash)
        _localize(id_iu, half, trash)
        for idst, oref in ((id_ui, o_ui), (id_iu, o_iu)):
            _zero_acc(zbuf, acc, sid, zr)
            plsc.subcore_barrier()

            def bb(b, carry, idst=idst):
                pltpu.sync_copy(onesb, acc.at[idst.at[b]], add=True)
                return carry
            lax.fori_loop(0, nb, bb, 0)
            plsc.subcore_barrier()
            pltpu.sync_copy(
                acc.at[pl.ds(sid * fl, fl)],
                oref.at[pl.ds(cid * half + sid * fl, fl)])
            plsc.subcore_barrier()

    out_t = tuple(jax.ShapeDtypeStruct((2 * half, CW), jnp.float32)
                  for _ in range(2))
    scratch = [
        pltpu.VMEM((nb, B), jnp.int32),
        pltpu.VMEM((nb, B), jnp.int32),
        pltpu.VMEM((B, CW), jnp.float32),
        pltpu.VMEM((B, CW), jnp.float32),
        pltpu.VMEM_SHARED((accr, CW), jnp.float32),
    ]
    return pl.kernel(body, out_type=out_t, mesh=_mesh(),
                     scratch_types=scratch,
                     compiler_params=_params())(dst3_ui, dst3_iu)


def _tc_chunked_matmul(x, w, bn=1000):
    n = x.shape[0]

    def body(x_ref, w_ref, *o_refs):
        y = jnp.dot(x_ref[...], w_ref[...],
                    preferred_element_type=jnp.float32)
        for c in range(4):
            o_refs[c][...] = y[:, c * CW:(c + 1) * CW]

    return pl.pallas_call(
        body,
        grid=(n // bn,),
        in_specs=[pl.BlockSpec((bn, D), lambda i: (i, 0)),
                  pl.BlockSpec((D, D), lambda i: (0, 0))],
        out_specs=[pl.BlockSpec((bn, CW), lambda i: (i, 0))] * 4,
        out_shape=[jax.ShapeDtypeStruct((n, CW), jnp.float32)] * 4,
    )(x, w)


def _tc_layer1(aggs, cnt, x, wc, bc, wl2, bn=1000):
    n = x.shape[0]

    def body(a0, a1, a2, a3, c_ref, x_ref, wc_ref, bc_ref, wl2_ref,
             h_ref, *y_refs):
        agg = jnp.concatenate(
            [a0[...], a1[...], a2[...], a3[...]], axis=1)
        recip = 1.0 / jnp.clip(c_ref[:, 0:1], 1.0, None)
        h = jnp.maximum(
            agg * recip
            + jnp.dot(x_ref[...], wc_ref[...],
                      preferred_element_type=jnp.float32)
            + bc_ref[...], 0.0)
        h_ref[...] = h
        y2 = jnp.dot(h, wl2_ref[...], preferred_element_type=jnp.float32)
        for c in range(4):
            y_refs[c][...] = y2[:, c * CW:(c + 1) * CW]

    return pl.pallas_call(
        body,
        grid=(n // bn,),
        in_specs=[pl.BlockSpec((bn, CW), lambda i: (i, 0))] * 4
        + [pl.BlockSpec((bn, CW), lambda i: (i, 0)),
           pl.BlockSpec((bn, D), lambda i: (i, 0)),
           pl.BlockSpec((D, D), lambda i: (0, 0)),
           pl.BlockSpec((1, D), lambda i: (0, 0)),
           pl.BlockSpec((D, D), lambda i: (0, 0))],
        out_specs=[pl.BlockSpec((bn, D), lambda i: (i, 0))]
        + [pl.BlockSpec((bn, CW), lambda i: (i, 0))] * 4,
        out_shape=[jax.ShapeDtypeStruct((n, D), jnp.float32)]
        + [jax.ShapeDtypeStruct((n, CW), jnp.float32)] * 4,
    )(*aggs, cnt, x, wc, bc, wl2)


def _tc_layer2(aggs, cnt, h, we, be, bn=1000):
    n = h.shape[0]

    def body(a0, a1, a2, a3, c_ref, h_ref, we_ref, be_ref, o_ref):
        agg = jnp.concatenate(
            [a0[...], a1[...], a2[...], a3[...]], axis=1)
        recip = 1.0 / jnp.clip(c_ref[:, 0:1], 1.0, None)
        o_ref[...] = (agg * recip
                      + jnp.dot(h_ref[...], we_ref[...],
                                preferred_element_type=jnp.float32)
                      + be_ref[...])

    return pl.pallas_call(
        body,
        grid=(n // bn,),
        in_specs=[pl.BlockSpec((bn, CW), lambda i: (i, 0))] * 4
        + [pl.BlockSpec((bn, CW), lambda i: (i, 0)),
           pl.BlockSpec((bn, D), lambda i: (i, 0)),
           pl.BlockSpec((D, D), lambda i: (0, 0)),
           pl.BlockSpec((1, D), lambda i: (0, 0))],
        out_specs=pl.BlockSpec((bn, D), lambda i: (i, 0)),
        out_shape=jax.ShapeDtypeStruct((n, D), jnp.float32),
    )(*aggs, cnt, h, we, be)


def _prep_edges(ei):
    e = ei.shape[1]
    nb = -(-e // (NTILES * B))
    ep = NTILES * nb * B
    src = ei[0].astype(jnp.int32)
    dst = ei[1].astype(jnp.int32)
    if ep > e:
        pad = ep - e
        src = jnp.concatenate([src, jnp.zeros((pad,), jnp.int32)])
        dst = jnp.concatenate(
            [dst, jnp.full((pad,), 1 << 29, jnp.int32)])
    return src.reshape(NTILES, nb, B), dst.reshape(NTILES, nb, B)


def kernel(x_user, x_item, edge_index_ui, edge_index_iu,
           conv1_ui_Wl, conv1_ui_bl, conv1_ui_Wr,
           conv1_iu_Wl, conv1_iu_bl, conv1_iu_Wr,
           lin1_user_W, lin1_user_b,
           lin1_item_W, lin1_item_b,
           conv2_ui_Wl, conv2_ui_bl, conv2_ui_Wr,
           conv2_iu_Wl, conv2_iu_bl, conv2_iu_Wr,
           lin2_user_W, lin2_user_b,
           lin2_item_W, lin2_item_b):
    n_user = x_user.shape[0]
    n_item = x_item.shape[0]
    assert n_user == n_item
    n = n_user
    half = -(-n // (2 * NTILES * B)) * (NTILES * B)

    wc_u = conv1_iu_Wr + lin1_user_W
    bc_u = (conv1_iu_bl + lin1_user_b).reshape(1, D)
    wc_i = conv1_ui_Wr + lin1_item_W
    bc_i = (conv1_ui_bl + lin1_item_b).reshape(1, D)
    we_u = conv2_iu_Wr + lin2_user_W
    be_u = (conv2_iu_bl + lin2_user_b).reshape(1, D)
    we_i = conv2_ui_Wr + lin2_item_W
    be_i = (conv2_ui_bl + lin2_item_b).reshape(1, D)

    src3_ui, dst3_ui = _prep_edges(edge_index_ui)
    src3_iu, dst3_iu = _prep_edges(edge_index_iu)

    y1u = _tc_chunked_matmul(x_user, conv1_ui_Wl)
    y1i = _tc_chunked_matmul(x_item, conv1_iu_Wl)
    cnt_ui, cnt_iu = _sc_counts(dst3_ui, dst3_iu, half)
    agg_ui1 = _sc_seg_sum(y1u, src3_ui, dst3_ui, half)
    agg_iu1 = _sc_seg_sum(y1i, src3_iu, dst3_iu, half)

    h1_user, *y2u = _tc_layer1(agg_iu1, cnt_iu, x_user, wc_u, bc_u,
                               conv2_ui_Wl)
    h1_item, *y2i = _tc_layer1(agg_ui1, cnt_ui, x_item, wc_i, bc_i,
                               conv2_iu_Wl)

    agg_ui2 = _sc_seg_sum(y2u, src3_ui, dst3_ui, half)
    agg_iu2 = _sc_seg_sum(y2i, src3_iu, dst3_iu, half)

    out_user = _tc_layer2(agg_iu2, cnt_iu, h1_user, we_u, be_u)
    out_item = _tc_layer2(agg_ui2, cnt_ui, h1_item, we_i, be_i)
    return (out_user, out_item)


# speedup vs baseline: 1.4584x; 1.0243x over previous
import jax
import jax.numpy as jnp
from jax import lax
from jax.experimental import pallas as pl
from jax.experimental.pallas import tpu as pltpu
from jax.experimental.pallas import tpu_sc as plsc

D = 128
CW = 32
NCORES = 2
NTILES = 16
B = 128


def _mesh():
    return plsc.VectorSubcoreMesh(
        core_axis_name="c", subcore_axis_name="s",
        num_cores=NCORES, num_subcores=NTILES)


def _params():
    return pltpu.CompilerParams(use_tc_tiling_on_sc=False)


def _fill(buf, val):
    v = jnp.full((16,), val, jnp.float32)

    def row(i, carry):
        buf[i, pl.ds(0, 16)] = v
        buf[i, pl.ds(16, 16)] = v
        return carry

    lax.fori_loop(0, buf.shape[0], row, 0)


def _localize(idst, half, trash):
    nb = idst.shape[0]
    base = lax.axis_index("c") * half
    tr = jnp.full((16,), trash, jnp.int32)

    def row(b, carry):
        for v in range(B // 16):
            x = idst[b, pl.ds(v * 16, 16)]
            l = x - jnp.full((16,), base, jnp.int32)
            keep = (l >= 0) & (l < half)
            idst[b, pl.ds(v * 16, 16)] = jnp.where(keep, l, tr)
        return carry

    lax.fori_loop(0, nb, row, 0)


def _zero_acc(zbuf, acc, sid, zrows):
    def zr(j, carry):
        pltpu.sync_copy(zbuf, acc.at[pl.ds(sid * zrows + j * B, B)])
        return carry
    lax.fori_loop(0, zrows // B, zr, 0)


def _sc_seg_sum(ys, src3, dst3, half):
    nb = src3.shape[1]
    accr = half + NTILES * B
    trash = half + NTILES * B // 2
    fl = half // NTILES
    zr = accr // NTILES

    def body(y0, y1, y2, y3, srcr, dstr, o0, o1, o2, o3,
             isrc, idst, rows0, rows1, zbuf, acc, gs0, gs1):
        cid = lax.axis_index("c")
        sid = lax.axis_index("s")
        _fill(zbuf, 0.0)
        pltpu.sync_copy(srcr.at[sid], isrc)
        pltpu.sync_copy(dstr.at[sid], idst)
        _localize(idst, half, trash)
        yrefs = (y0, y1, y2, y3)
        orefs = (o0, o1, o2, o3)
        for p in range(4):
            _zero_acc(zbuf, acc, sid, zr)
            plsc.subcore_barrier()
            y = yrefs[p]
            pltpu.async_copy(y.at[isrc.at[0]], rows0, gs0)

            def bb(i, carry):
                b0 = 2 * i
                b1 = b0 + 1
                pltpu.async_copy(y.at[isrc.at[b1]], rows1, gs1)
                pltpu.make_async_copy(y.at[isrc.at[b0]], rows0, gs0).wait()
                pltpu.sync_copy(rows0, acc.at[idst.at[b0]], add=True)

                @pl.when(i < nb // 2 - 1)
                def _():
                    pltpu.async_copy(y.at[isrc.at[b0 + 2]], rows0, gs0)
                pltpu.make_async_copy(y.at[isrc.at[b1]], rows1, gs1).wait()
                pltpu.sync_copy(rows1, acc.at[idst.at[b1]], add=True)
                return carry
            lax.fori_loop(0, nb // 2, bb, 0)
            plsc.subcore_barrier()
            pltpu.sync_copy(
                acc.at[pl.ds(sid * fl, fl)],
                orefs[p].at[pl.ds(cid * half + sid * fl, fl)])
            plsc.subcore_barrier()

    out_t = tuple(jax.ShapeDtypeStruct((2 * half, CW), jnp.float32)
                  for _ in range(4))
    scratch = [
        pltpu.VMEM((nb, B), jnp.int32),
        pltpu.VMEM((nb, B), jnp.int32),
        pltpu.VMEM((B, CW), jnp.float32),
        pltpu.VMEM((B, CW), jnp.float32),
        pltpu.VMEM((B, CW), jnp.float32),
        pltpu.VMEM_SHARED((accr, CW), jnp.float32),
        pltpu.SemaphoreType.DMA,
        pltpu.SemaphoreType.DMA,
    ]
    return pl.kernel(body, out_type=out_t, mesh=_mesh(),
                     scratch_types=scratch,
                     compiler_params=_params())(*ys, src3, dst3)


def _sc_counts(dst3_ui, dst3_iu, half):
    nb = dst3_ui.shape[1]
    accr = half + NTILES * B
    trash = half + NTILES * B // 2
    fl = half // NTILES
    zr = accr // NTILES

    k = next(d for d in (16, 14, 12, 8, 7, 4, 2, 1) if nb % d == 0)

    def body(dui, diu, o_ui, o_iu, id_ui, id_iu, onesb, zbuf, acc, sem):
        cid = lax.axis_index("c")
        sid = lax.axis_index("s")
        _fill(onesb, 1.0)
        _fill(zbuf, 0.0)
        pltpu.sync_copy(dui.at[sid], id_ui)
        pltpu.sync_copy(diu.at[sid], id_iu)
        _localize(id_ui, half, trash)
        _localize(id_iu, half, trash)
        for idst, oref in ((id_ui, o_ui), (id_iu, o_iu)):
            _zero_acc(zbuf, acc, sid, zr)
            plsc.subcore_barrier()

            def bb(i, carry, idst=idst):
                for j in range(k):
                    pltpu.async_copy(onesb, acc.at[idst.at[i * k + j]], sem,
                                     add=True)
                for j in range(k):
                    pltpu.make_async_copy(
                        onesb, acc.at[idst.at[i * k + j]], sem).wait()
                return carry
            lax.fori_loop(0, nb // k, bb, 0)
            plsc.subcore_barrier()
            pltpu.sync_copy(
                acc.at[pl.ds(sid * fl, fl)],
                oref.at[pl.ds(cid * half + sid * fl, fl)])
            plsc.subcore_barrier()

    out_t = tuple(jax.ShapeDtypeStruct((2 * half, CW), jnp.float32)
                  for _ in range(2))
    scratch = [
        pltpu.VMEM((nb, B), jnp.int32),
        pltpu.VMEM((nb, B), jnp.int32),
        pltpu.VMEM((B, CW), jnp.float32),
        pltpu.VMEM((B, CW), jnp.float32),
        pltpu.VMEM_SHARED((accr, CW), jnp.float32),
        pltpu.SemaphoreType.DMA,
    ]
    return pl.kernel(body, out_type=out_t, mesh=_mesh(),
                     scratch_types=scratch,
                     compiler_params=_params())(dst3_ui, dst3_iu)


def _tc_chunked_matmul(x, w, bn=1000):
    n = x.shape[0]

    def body(x_ref, w_ref, *o_refs):
        y = jnp.dot(x_ref[...], w_ref[...],
                    preferred_element_type=jnp.float32)
        for c in range(4):
            o_refs[c][...] = y[:, c * CW:(c + 1) * CW]

    return pl.pallas_call(
        body,
        grid=(n // bn,),
        in_specs=[pl.BlockSpec((bn, D), lambda i: (i, 0)),
                  pl.BlockSpec((D, D), lambda i: (0, 0))],
        out_specs=[pl.BlockSpec((bn, CW), lambda i: (i, 0))] * 4,
        out_shape=[jax.ShapeDtypeStruct((n, CW), jnp.float32)] * 4,
    )(x, w)


def _tc_layer1(aggs, cnt, x, wc, bc, wl2, bn=1000):
    n = x.shape[0]

    def body(a0, a1, a2, a3, c_ref, x_ref, wc_ref, bc_ref, wl2_ref,
             h_ref, *y_refs):
        agg = jnp.concatenate(
            [a0[...], a1[...], a2[...], a3[...]], axis=1)
        recip = 1.0 / jnp.clip(c_ref[:, 0:1], 1.0, None)
        h = jnp.maximum(
            agg * recip
            + jnp.dot(x_ref[...], wc_ref[...],
                      preferred_element_type=jnp.float32)
            + bc_ref[...], 0.0)
        h_ref[...] = h
        y2 = jnp.dot(h, wl2_ref[...], preferred_element_type=jnp.float32)
        for c in range(4):
            y_refs[c][...] = y2[:, c * CW:(c + 1) * CW]

    return pl.pallas_call(
        body,
        grid=(n // bn,),
        in_specs=[pl.BlockSpec((bn, CW), lambda i: (i, 0))] * 4
        + [pl.BlockSpec((bn, CW), lambda i: (i, 0)),
           pl.BlockSpec((bn, D), lambda i: (i, 0)),
           pl.BlockSpec((D, D), lambda i: (0, 0)),
           pl.BlockSpec((1, D), lambda i: (0, 0)),
           pl.BlockSpec((D, D), lambda i: (0, 0))],
        out_specs=[pl.BlockSpec((bn, D), lambda i: (i, 0))]
        + [pl.BlockSpec((bn, CW), lambda i: (i, 0))] * 4,
        out_shape=[jax.ShapeDtypeStruct((n, D), jnp.float32)]
        + [jax.ShapeDtypeStruct((n, CW), jnp.float32)] * 4,
    )(*aggs, cnt, x, wc, bc, wl2)


def _tc_layer2(aggs, cnt, h, we, be, bn=1000):
    n = h.shape[0]

    def body(a0, a1, a2, a3, c_ref, h_ref, we_ref, be_ref, o_ref):
        agg = jnp.concatenate(
            [a0[...], a1[...], a2[...], a3[...]], axis=1)
        recip = 1.0 / jnp.clip(c_ref[:, 0:1], 1.0, None)
        o_ref[...] = (agg * recip
                      + jnp.dot(h_ref[...], we_ref[...],
                                preferred_element_type=jnp.float32)
                      + be_ref[...])

    return pl.pallas_call(
        body,
        grid=(n // bn,),
        in_specs=[pl.BlockSpec((bn, CW), lambda i: (i, 0))] * 4
        + [pl.BlockSpec((bn, CW), lambda i: (i, 0)),
           pl.BlockSpec((bn, D), lambda i: (i, 0)),
           pl.BlockSpec((D, D), lambda i: (0, 0)),
           pl.BlockSpec((1, D), lambda i: (0, 0))],
        out_specs=pl.BlockSpec((bn, D), lambda i: (i, 0)),
        out_shape=jax.ShapeDtypeStruct((n, D), jnp.float32),
    )(*aggs, cnt, h, we, be)


def _prep_edges(ei):
    e = ei.shape[1]
    nb = -(-e // (NTILES * B))
    ep = NTILES * nb * B
    src = ei[0].astype(jnp.int32)
    dst = ei[1].astype(jnp.int32)
    if ep > e:
        pad = ep - e
        src = jnp.concatenate([src, jnp.zeros((pad,), jnp.int32)])
        dst = jnp.concatenate(
            [dst, jnp.full((pad,), 1 << 29, jnp.int32)])
    return src.reshape(NTILES, nb, B), dst.reshape(NTILES, nb, B)


def kernel(x_user, x_item, edge_index_ui, edge_index_iu,
           conv1_ui_Wl, conv1_ui_bl, conv1_ui_Wr,
           conv1_iu_Wl, conv1_iu_bl, conv1_iu_Wr,
           lin1_user_W, lin1_user_b,
           lin1_item_W, lin1_item_b,
           conv2_ui_Wl, conv2_ui_bl, conv2_ui_Wr,
           conv2_iu_Wl, conv2_iu_bl, conv2_iu_Wr,
           lin2_user_W, lin2_user_b,
           lin2_item_W, lin2_item_b):
    n_user = x_user.shape[0]
    n_item = x_item.shape[0]
    assert n_user == n_item
    n = n_user
    half = -(-n // (2 * NTILES * B)) * (NTILES * B)

    wc_u = conv1_iu_Wr + lin1_user_W
    bc_u = (conv1_iu_bl + lin1_user_b).reshape(1, D)
    wc_i = conv1_ui_Wr + lin1_item_W
    bc_i = (conv1_ui_bl + lin1_item_b).reshape(1, D)
    we_u = conv2_iu_Wr + lin2_user_W
    be_u = (conv2_iu_bl + lin2_user_b).reshape(1, D)
    we_i = conv2_ui_Wr + lin2_item_W
    be_i = (conv2_ui_bl + lin2_item_b).reshape(1, D)

    src3_ui, dst3_ui = _prep_edges(edge_index_ui)
    src3_iu, dst3_iu = _prep_edges(edge_index_iu)

    y1u = _tc_chunked_matmul(x_user, conv1_ui_Wl)
    y1i = _tc_chunked_matmul(x_item, conv1_iu_Wl)
    cnt_ui, cnt_iu = _sc_counts(dst3_ui, dst3_iu, half)
    agg_ui1 = _sc_seg_sum(y1u, src3_ui, dst3_ui, half)
    agg_iu1 = _sc_seg_sum(y1i, src3_iu, dst3_iu, half)

    h1_user, *y2u = _tc_layer1(agg_iu1, cnt_iu, x_user, wc_u, bc_u,
                               conv2_ui_Wl)
    h1_item, *y2i = _tc_layer1(agg_ui1, cnt_ui, x_item, wc_i, bc_i,
                               conv2_iu_Wl)

    agg_ui2 = _sc_seg_sum(y2u, src3_ui, dst3_ui, half)
    agg_iu2 = _sc_seg_sum(y2i, src3_iu, dst3_iu, half)

    out_user = _tc_layer2(agg_iu2, cnt_iu, h1_user, we_u, be_u)
    out_item = _tc_layer2(agg_ui2, cnt_ui, h1_item, we_i, be_i)
    return (out_user, out_item)


# speedup vs baseline: 1.4600x; 1.0011x over previous
import jax
import jax.numpy as jnp
from jax import lax
from jax.experimental import pallas as pl
from jax.experimental.pallas import tpu as pltpu
from jax.experimental.pallas import tpu_sc as plsc

D = 128
CW = 32
NCORES = 2
NTILES = 16
B = 128


def _mesh():
    return plsc.VectorSubcoreMesh(
        core_axis_name="c", subcore_axis_name="s",
        num_cores=NCORES, num_subcores=NTILES)


def _params():
    return pltpu.CompilerParams(use_tc_tiling_on_sc=False)


def _fill(buf, val):
    v = jnp.full((16,), val, jnp.float32)

    def row(i, carry):
        buf[i, pl.ds(0, 16)] = v
        buf[i, pl.ds(16, 16)] = v
        return carry

    lax.fori_loop(0, buf.shape[0], row, 0)


def _localize(idst, half, trash):
    nb = idst.shape[0]
    base = lax.axis_index("c") * half
    tr = jnp.full((16,), trash, jnp.int32)

    def row(b, carry):
        for v in range(B // 16):
            x = idst[b, pl.ds(v * 16, 16)]
            l = x - jnp.full((16,), base, jnp.int32)
            keep = (l >= 0) & (l < half)
            idst[b, pl.ds(v * 16, 16)] = jnp.where(keep, l, tr)
        return carry

    lax.fori_loop(0, nb, row, 0)


def _zero_acc(zbuf, acc, sid, zrows):
    def zr(j, carry):
        pltpu.sync_copy(zbuf, acc.at[pl.ds(sid * zrows + j * B, B)])
        return carry
    lax.fori_loop(0, zrows // B, zr, 0)


def _sc_seg_sum(ys, src3, dst3, half):
    nb = src3.shape[1]
    accr = half + NTILES * B
    trash = half + NTILES * B // 2
    fl = half // NTILES
    zr = accr // NTILES

    nbuf = next(d for d in (4, 7, 2, 1) if nb % d == 0)

    def body(y0, y1, y2, y3, srcr, dstr, o0, o1, o2, o3,
             isrc, idst, r0, r1, r2, r3, zbuf, acc,
             g0, g1, g2, g3, s0, s1, s2, s3):
        cid = lax.axis_index("c")
        sid = lax.axis_index("s")
        rows = (r0, r1, r2, r3)[:nbuf]
        gs = (g0, g1, g2, g3)[:nbuf]
        ss = (s0, s1, s2, s3)[:nbuf]
        _fill(zbuf, 0.0)
        pltpu.sync_copy(srcr.at[sid], isrc)
        pltpu.sync_copy(dstr.at[sid], idst)
        _localize(idst, half, trash)
        yrefs = (y0, y1, y2, y3)
        orefs = (o0, o1, o2, o3)
        for p in range(4):
            _zero_acc(zbuf, acc, sid, zr)
            plsc.subcore_barrier()
            y = yrefs[p]
            for j in range(nbuf):
                pltpu.async_copy(y.at[isrc.at[j]], rows[j], gs[j])

            def bb(i, carry):
                for j in range(nbuf):
                    b = i * nbuf + j
                    pltpu.make_async_copy(
                        y.at[isrc.at[b]], rows[j], gs[j]).wait()
                    pltpu.async_copy(rows[j], acc.at[idst.at[b]], ss[j],
                                     add=True)
                for j in range(nbuf):
                    b = i * nbuf + j
                    pltpu.make_async_copy(
                        rows[j], acc.at[idst.at[b]], ss[j]).wait()

                    @pl.when(i < nb // nbuf - 1)
                    def _():
                        pltpu.async_copy(
                            y.at[isrc.at[b + nbuf]], rows[j], gs[j])
                return carry
            lax.fori_loop(0, nb // nbuf, bb, 0)
            plsc.subcore_barrier()
            pltpu.sync_copy(
                acc.at[pl.ds(sid * fl, fl)],
                orefs[p].at[pl.ds(cid * half + sid * fl, fl)])
            plsc.subcore_barrier()

    out_t = tuple(jax.ShapeDtypeStruct((2 * half, CW), jnp.float32)
                  for _ in range(4))
    scratch = [
        pltpu.VMEM((nb, B), jnp.int32),
        pltpu.VMEM((nb, B), jnp.int32),
        pltpu.VMEM((B, CW), jnp.float32),
        pltpu.VMEM((B, CW), jnp.float32),
        pltpu.VMEM((B, CW), jnp.float32),
        pltpu.VMEM((B, CW), jnp.float32),
        pltpu.VMEM((B, CW), jnp.float32),
        pltpu.VMEM_SHARED((accr, CW), jnp.float32),
        pltpu.SemaphoreType.DMA,
        pltpu.SemaphoreType.DMA,
        pltpu.SemaphoreType.DMA,
        pltpu.SemaphoreType.DMA,
        pltpu.SemaphoreType.DMA,
        pltpu.SemaphoreType.DMA,
        pltpu.SemaphoreType.DMA,
        pltpu.SemaphoreType.DMA,
    ]
    return pl.kernel(body, out_type=out_t, mesh=_mesh(),
                     scratch_types=scratch,
                     compiler_params=_params())(*ys, src3, dst3)


def _sc_counts(dst3_ui, dst3_iu, half):
    nb = dst3_ui.shape[1]
    accr = half + NTILES * B
    trash = half + NTILES * B // 2
    fl = half // NTILES
    zr = accr // NTILES

    k = next(d for d in (16, 14, 12, 8, 7, 4, 2, 1) if nb % d == 0)

    def body(dui, diu, o_ui, o_iu, id_ui, id_iu, onesb, zbuf, acc, sem):
        cid = lax.axis_index("c")
        sid = lax.axis_index("s")
        _fill(onesb, 1.0)
        _fill(zbuf, 0.0)
        pltpu.sync_copy(dui.at[sid], id_ui)
        pltpu.sync_copy(diu.at[sid], id_iu)
        _localize(id_ui, half, trash)
        _localize(id_iu, half, trash)
        for idst, oref in ((id_ui, o_ui), (id_iu, o_iu)):
            _zero_acc(zbuf, acc, sid, zr)
            plsc.subcore_barrier()

            def bb(i, carry, idst=idst):
                for j in range(k):
                    pltpu.async_copy(onesb, acc.at[idst.at[i * k + j]], sem,
                                     add=True)
                for j in range(k):
                    pltpu.make_async_copy(
                        onesb, acc.at[idst.at[i * k + j]], sem).wait()
                return carry
            lax.fori_loop(0, nb // k, bb, 0)
            plsc.subcore_barrier()
            pltpu.sync_copy(
                acc.at[pl.ds(sid * fl, fl)],
                oref.at[pl.ds(cid * half + sid * fl, fl)])
            plsc.subcore_barrier()

    out_t = tuple(jax.ShapeDtypeStruct((2 * half, CW), jnp.float32)
                  for _ in range(2))
    scratch = [
        pltpu.VMEM((nb, B), jnp.int32),
        pltpu.VMEM((nb, B), jnp.int32),
        pltpu.VMEM((B, CW), jnp.float32),
        pltpu.VMEM((B, CW), jnp.float32),
        pltpu.VMEM_SHARED((accr, CW), jnp.float32),
        pltpu.SemaphoreType.DMA,
    ]
    return pl.kernel(body, out_type=out_t, mesh=_mesh(),
                     scratch_types=scratch,
                     compiler_params=_params())(dst3_ui, dst3_iu)


def _tc_chunked_matmul(x, w, bn=1000):
    n = x.shape[0]

    def body(x_ref, w_ref, *o_refs):
        y = jnp.dot(x_ref[...], w_ref[...],
                    preferred_element_type=jnp.float32)
        for c in range(4):
            o_refs[c][...] = y[:, c * CW:(c + 1) * CW]

    return pl.pallas_call(
        body,
        grid=(n // bn,),
        in_specs=[pl.BlockSpec((bn, D), lambda i: (i, 0)),
                  pl.BlockSpec((D, D), lambda i: (0, 0))],
        out_specs=[pl.BlockSpec((bn, CW), lambda i: (i, 0))] * 4,
        out_shape=[jax.ShapeDtypeStruct((n, CW), jnp.float32)] * 4,
    )(x, w)


def _tc_layer1(aggs, cnt, x, wc, bc, wl2, bn=1000):
    n = x.shape[0]

    def body(a0, a1, a2, a3, c_ref, x_ref, wc_ref, bc_ref, wl2_ref,
             h_ref, *y_refs):
        agg = jnp.concatenate(
            [a0[...], a1[...], a2[...], a3[...]], axis=1)
        recip = 1.0 / jnp.clip(c_ref[:, 0:1], 1.0, None)
        h = jnp.maximum(
            agg * recip
            + jnp.dot(x_ref[...], wc_ref[...],
                      preferred_element_type=jnp.float32)
            + bc_ref[...], 0.0)
        h_ref[...] = h
        y2 = jnp.dot(h, wl2_ref[...], preferred_element_type=jnp.float32)
        for c in range(4):
            y_refs[c][...] = y2[:, c * CW:(c + 1) * CW]

    return pl.pallas_call(
        body,
        grid=(n // bn,),
        in_specs=[pl.BlockSpec((bn, CW), lambda i: (i, 0))] * 4
        + [pl.BlockSpec((bn, CW), lambda i: (i, 0)),
           pl.BlockSpec((bn, D), lambda i: (i, 0)),
           pl.BlockSpec((D, D), lambda i: (0, 0)),
           pl.BlockSpec((1, D), lambda i: (0, 0)),
           pl.BlockSpec((D, D), lambda i: (0, 0))],
        out_specs=[pl.BlockSpec((bn, D), lambda i: (i, 0))]
        + [pl.BlockSpec((bn, CW), lambda i: (i, 0))] * 4,
        out_shape=[jax.ShapeDtypeStruct((n, D), jnp.float32)]
        + [jax.ShapeDtypeStruct((n, CW), jnp.float32)] * 4,
    )(*aggs, cnt, x, wc, bc, wl2)


def _tc_layer2(aggs, cnt, h, we, be, bn=1000):
    n = h.shape[0]

    def body(a0, a1, a2, a3, c_ref, h_ref, we_ref, be_ref, o_ref):
        agg = jnp.concatenate(
            [a0[...], a1[...], a2[...], a3[...]], axis=1)
        recip = 1.0 / jnp.clip(c_ref[:, 0:1], 1.0, None)
        o_ref[...] = (agg * recip
                      + jnp.dot(h_ref[...], we_ref[...],
                                preferred_element_type=jnp.float32)
                      + be_ref[...])

    return pl.pallas_call(
        body,
        grid=(n // bn,),
        in_specs=[pl.BlockSpec((bn, CW), lambda i: (i, 0))] * 4
        + [pl.BlockSpec((bn, CW), lambda i: (i, 0)),
           pl.BlockSpec((bn, D), lambda i: (i, 0)),
           pl.BlockSpec((D, D), lambda i: (0, 0)),
           pl.BlockSpec((1, D), lambda i: (0, 0))],
        out_specs=pl.BlockSpec((bn, D), lambda i: (i, 0)),
        out_shape=jax.ShapeDtypeStruct((n, D), jnp.float32),
    )(*aggs, cnt, h, we, be)


def _prep_edges(ei):
    e = ei.shape[1]
    nb = -(-e // (NTILES * B))
    ep = NTILES * nb * B
    src = ei[0].astype(jnp.int32)
    dst = ei[1].astype(jnp.int32)
    if ep > e:
        pad = ep - e
        src = jnp.concatenate([src, jnp.zeros((pad,), jnp.int32)])
        dst = jnp.concatenate(
            [dst, jnp.full((pad,), 1 << 29, jnp.int32)])
    return src.reshape(NTILES, nb, B), dst.reshape(NTILES, nb, B)


def kernel(x_user, x_item, edge_index_ui, edge_index_iu,
           conv1_ui_Wl, conv1_ui_bl, conv1_ui_Wr,
           conv1_iu_Wl, conv1_iu_bl, conv1_iu_Wr,
           lin1_user_W, lin1_user_b,
           lin1_item_W, lin1_item_b,
           conv2_ui_Wl, conv2_ui_bl, conv2_ui_Wr,
           conv2_iu_Wl, conv2_iu_bl, conv2_iu_Wr,
           lin2_user_W, lin2_user_b,
           lin2_item_W, lin2_item_b):
    n_user = x_user.shape[0]
    n_item = x_item.shape[0]
    assert n_user == n_item
    n = n_user
    half = -(-n // (2 * NTILES * B)) * (NTILES * B)

    wc_u = conv1_iu_Wr + lin1_user_W
    bc_u = (conv1_iu_bl + lin1_user_b).reshape(1, D)
    wc_i = conv1_ui_Wr + lin1_item_W
    bc_i = (conv1_ui_bl + lin1_item_b).reshape(1, D)
    we_u = conv2_iu_Wr + lin2_user_W
    be_u = (conv2_iu_bl + lin2_user_b).reshape(1, D)
    we_i = conv2_ui_Wr + lin2_item_W
    be_i = (conv2_ui_bl + lin2_item_b).reshape(1, D)

    src3_ui, dst3_ui = _prep_edges(edge_index_ui)
    src3_iu, dst3_iu = _prep_edges(edge_index_iu)

    y1u = _tc_chunked_matmul(x_user, conv1_ui_Wl)
    y1i = _tc_chunked_matmul(x_item, conv1_iu_Wl)
    cnt_ui, cnt_iu = _sc_counts(dst3_ui, dst3_iu, half)
    agg_ui1 = _sc_seg_sum(y1u, src3_ui, dst3_ui, half)
    agg_iu1 = _sc_seg_sum(y1i, src3_iu, dst3_iu, half)

    h1_user, *y2u = _tc_layer1(agg_iu1, cnt_iu, x_user, wc_u, bc_u,
                               conv2_ui_Wl)
    h1_item, *y2i = _tc_layer1(agg_ui1, cnt_ui, x_item, wc_i, bc_i,
                               conv2_iu_Wl)

    agg_ui2 = _sc_seg_sum(y2u, src3_ui, dst3_ui, half)
    agg_iu2 = _sc_seg_sum(y2i, src3_iu, dst3_iu, half)

    out_user = _tc_layer2(agg_iu2, cnt_iu, h1_user, we_u, be_u)
    out_item = _tc_layer2(agg_ui2, cnt_ui, h1_item, we_i, be_i)
    return (out_user, out_item)


# speedup vs baseline: 2.8206x; 1.9318x over previous
import jax
import jax.numpy as jnp
from jax import lax
from jax.experimental import pallas as pl
from jax.experimental.pallas import tpu as pltpu
from jax.experimental.pallas import tpu_sc as plsc

D = 128
CW = 32
NCORES = 2
NTILES = 16
B = 128


def _mesh():
    return plsc.VectorSubcoreMesh(
        core_axis_name="c", subcore_axis_name="s",
        num_cores=NCORES, num_subcores=NTILES)


def _params():
    return pltpu.CompilerParams(use_tc_tiling_on_sc=False)


def _fill(buf, val):
    v = jnp.full((16,), val, jnp.float32)

    def row(i, carry):
        buf[i, pl.ds(0, 16)] = v
        buf[i, pl.ds(16, 16)] = v
        return carry

    lax.fori_loop(0, buf.shape[0], row, 0)


def _localize(idst, half, trash):
    nb = idst.shape[0]
    base = lax.axis_index("c") * half
    tr = jnp.full((16,), trash, jnp.int32)

    def row(b, carry):
        for v in range(B // 16):
            x = idst[b, pl.ds(v * 16, 16)]
            l = x - jnp.full((16,), base, jnp.int32)
            keep = (l >= 0) & (l < half)
            idst[b, pl.ds(v * 16, 16)] = jnp.where(keep, l, tr)
        return carry

    lax.fori_loop(0, nb, row, 0)


def _zero_acc(zbuf, acc, sid, zrows):
    def zr(j, carry):
        pltpu.sync_copy(zbuf, acc.at[pl.ds(sid * zrows + j * B, B)])
        return carry
    lax.fori_loop(0, zrows // B, zr, 0)


CH = 7


def _sc_seg_sum_dual(ys, src3, dst3, accr):
    nb = src3.shape[2]
    assert nb % (2 * CH) == 0
    nch = nb // CH
    zr = accr // NTILES

    def chunk(y, acc, ib, jb, rows, gs, ss):
        for b in range(4):
            pltpu.async_copy(y.at[ib.at[b]], rows[b], gs[b])
        for b in range(4):
            pltpu.make_async_copy(y.at[ib.at[b]], rows[b], gs[b]).wait()
            pltpu.async_copy(rows[b], acc.at[jb.at[b]], ss[b], add=True)
        for b in range(4, CH):
            s = b - 4
            pltpu.make_async_copy(rows[s], acc.at[jb.at[s]], ss[s]).wait()
            pltpu.async_copy(y.at[ib.at[b]], rows[s], gs[s])
        for b in range(4, CH):
            s = b - 4
            pltpu.make_async_copy(y.at[ib.at[b]], rows[s], gs[s]).wait()
            pltpu.async_copy(rows[s], acc.at[jb.at[b]], ss[s], add=True)
        pltpu.make_async_copy(rows[3], acc.at[jb.at[3]], ss[3]).wait()
        for b in range(4, CH):
            s = b - 4
            pltpu.make_async_copy(rows[s], acc.at[jb.at[b]], ss[s]).wait()

    def body(y0, y1, y2, y3, srcr, dstr,
             ou0, ou1, ou2, ou3, oi0, oi1, oi2, oi3,
             ibA, jbA, ibB, jbB, r0, r1, r2, r3, zbuf, acc,
             semA, semB, g0, g1, g2, g3, s0, s1, s2, s3):
        cid = lax.axis_index("c")
        sid = lax.axis_index("s")
        rows = (r0, r1, r2, r3)
        gs = (g0, g1, g2, g3)
        ss = (s0, s1, s2, s3)
        _fill(zbuf, 0.0)
        yrefs = (y0, y1, y2, y3)
        ou = (ou0, ou1, ou2, ou3)
        oi = (oi0, oi1, oi2, oi3)
        for p in range(4):
            pltpu.async_copy(srcr.at[cid, sid, pl.ds(0, CH)], ibA, semA)
            pltpu.async_copy(dstr.at[cid, sid, pl.ds(0, CH)], jbA, semA)
            _zero_acc(zbuf, acc, sid, zr)
            plsc.subcore_barrier()
            y = yrefs[p].at[cid]

            def pair(cp, carry):
                c0 = 2 * cp
                pltpu.make_async_copy(
                    srcr.at[cid, sid, pl.ds(c0 * CH, CH)], ibA, semA).wait()
                pltpu.make_async_copy(
                    dstr.at[cid, sid, pl.ds(c0 * CH, CH)], jbA, semA).wait()
                pltpu.async_copy(
                    srcr.at[cid, sid, pl.ds((c0 + 1) * CH, CH)], ibB, semB)
                pltpu.async_copy(
                    dstr.at[cid, sid, pl.ds((c0 + 1) * CH, CH)], jbB, semB)
                chunk(y, acc, ibA, jbA, rows, gs, ss)
                pltpu.make_async_copy(
                    srcr.at[cid, sid, pl.ds((c0 + 1) * CH, CH)], ibB,
                    semB).wait()
                pltpu.make_async_copy(
                    dstr.at[cid, sid, pl.ds((c0 + 1) * CH, CH)], jbB,
                    semB).wait()

                @pl.when(cp < nch // 2 - 1)
                def _():
                    pltpu.async_copy(
                        srcr.at[cid, sid, pl.ds((c0 + 2) * CH, CH)], ibA,
                        semA)
                    pltpu.async_copy(
                        dstr.at[cid, sid, pl.ds((c0 + 2) * CH, CH)], jbA,
                        semA)
                chunk(y, acc, ibB, jbB, rows, gs, ss)
                return carry
            lax.fori_loop(0, nch // 2, pair, 0)
            plsc.subcore_barrier()

            @pl.when(cid == 0)
            def _():
                pltpu.sync_copy(acc.at[pl.ds(sid * zr, zr)],
                                ou[p].at[pl.ds(sid * zr, zr)])

            @pl.when(cid == 1)
            def _():
                pltpu.sync_copy(acc.at[pl.ds(sid * zr, zr)],
                                oi[p].at[pl.ds(sid * zr, zr)])
            plsc.subcore_barrier()

    out_t = tuple(jax.ShapeDtypeStruct((accr, CW), jnp.float32)
                  for _ in range(8))
    scratch = [
        pltpu.VMEM((CH, B), jnp.int32),
        pltpu.VMEM((CH, B), jnp.int32),
        pltpu.VMEM((CH, B), jnp.int32),
        pltpu.VMEM((CH, B), jnp.int32),
        pltpu.VMEM((B, CW), jnp.float32),
        pltpu.VMEM((B, CW), jnp.float32),
        pltpu.VMEM((B, CW), jnp.float32),
        pltpu.VMEM((B, CW), jnp.float32),
        pltpu.VMEM((B, CW), jnp.float32),
        pltpu.VMEM_SHARED((accr, CW), jnp.float32),
    ] + [pltpu.SemaphoreType.DMA] * 10
    return pl.kernel(body, out_type=out_t, mesh=_mesh(),
                     scratch_types=scratch,
                     compiler_params=_params())(*ys, src3, dst3)


def _sc_counts(dst3_ui, dst3_iu, half):
    nb = dst3_ui.shape[1]
    accr = half + NTILES * B
    trash = half + NTILES * B // 2
    fl = half // NTILES
    zr = accr // NTILES

    k = next(d for d in (16, 14, 12, 8, 7, 4, 2, 1) if nb % d == 0)

    def body(dui, diu, o_ui, o_iu, id_ui, id_iu, onesb, zbuf, acc, sem):
        cid = lax.axis_index("c")
        sid = lax.axis_index("s")
        _fill(onesb, 1.0)
        _fill(zbuf, 0.0)
        pltpu.sync_copy(dui.at[sid], id_ui)
        pltpu.sync_copy(diu.at[sid], id_iu)
        _localize(id_ui, half, trash)
        _localize(id_iu, half, trash)
        for idst, oref in ((id_ui, o_ui), (id_iu, o_iu)):
            _zero_acc(zbuf, acc, sid, zr)
            plsc.subcore_barrier()

            def bb(i, carry, idst=idst):
                for j in range(k):
                    pltpu.async_copy(onesb, acc.at[idst.at[i * k + j]], sem,
                                     add=True)
                for j in range(k):
                    pltpu.make_async_copy(
                        onesb, acc.at[idst.at[i * k + j]], sem).wait()
                return carry
            lax.fori_loop(0, nb // k, bb, 0)
            plsc.subcore_barrier()
            pltpu.sync_copy(
                acc.at[pl.ds(sid * fl, fl)],
                oref.at[pl.ds(cid * half + sid * fl, fl)])
            plsc.subcore_barrier()

    out_t = tuple(jax.ShapeDtypeStruct((2 * half, CW), jnp.float32)
                  for _ in range(2))
    scratch = [
        pltpu.VMEM((nb, B), jnp.int32),
        pltpu.VMEM((nb, B), jnp.int32),
        pltpu.VMEM((B, CW), jnp.float32),
        pltpu.VMEM((B, CW), jnp.float32),
        pltpu.VMEM_SHARED((accr, CW), jnp.float32),
        pltpu.SemaphoreType.DMA,
    ]
    return pl.kernel(body, out_type=out_t, mesh=_mesh(),
                     scratch_types=scratch,
                     compiler_params=_params())(dst3_ui, dst3_iu)


def _tc_stacked_matmul(xcat, w0, w1, bn=1000):
    n2 = xcat.shape[0]

    def body(x_ref, w0_ref, w1_ref, *o_refs):
        pid = pl.program_id(0)
        w = jnp.where(pid < n2 // (2 * bn), w0_ref[...], w1_ref[...])
        y = jnp.dot(x_ref[...], w, preferred_element_type=jnp.float32)
        for c in range(4):
            o_refs[c][...] = y[:, c * CW:(c + 1) * CW]

    outs = pl.pallas_call(
        body,
        grid=(n2 // bn,),
        in_specs=[pl.BlockSpec((bn, D), lambda i: (i, 0)),
                  pl.BlockSpec((D, D), lambda i: (0, 0)),
                  pl.BlockSpec((D, D), lambda i: (0, 0))],
        out_specs=[pl.BlockSpec((bn, CW), lambda i: (i, 0))] * 4,
        out_shape=[jax.ShapeDtypeStruct((n2, CW), jnp.float32)] * 4,
    )(xcat, w0, w1)
    return [o.reshape(2, n2 // 2, CW) for o in outs]


def _tc_layer1(aggs, cnt, x, wc, bc, wl2, bn=1000):
    n = x.shape[0]

    def body(a0, a1, a2, a3, c_ref, x_ref, wc_ref, bc_ref, wl2_ref,
             h_ref, *y_refs):
        agg = jnp.concatenate(
            [a0[...], a1[...], a2[...], a3[...]], axis=1)
        recip = 1.0 / jnp.clip(c_ref[:, 0:1], 1.0, None)
        h = jnp.maximum(
            agg * recip
            + jnp.dot(x_ref[...], wc_ref[...],
                      preferred_element_type=jnp.float32)
            + bc_ref[...], 0.0)
        h_ref[...] = h
        y2 = jnp.dot(h, wl2_ref[...], preferred_element_type=jnp.float32)
        for c in range(4):
            y_refs[c][...] = y2[:, c * CW:(c + 1) * CW]

    return pl.pallas_call(
        body,
        grid=(n // bn,),
        in_specs=[pl.BlockSpec((bn, CW), lambda i: (i, 0))] * 4
        + [pl.BlockSpec((bn, CW), lambda i: (i, 0)),
           pl.BlockSpec((bn, D), lambda i: (i, 0)),
           pl.BlockSpec((D, D), lambda i: (0, 0)),
           pl.BlockSpec((1, D), lambda i: (0, 0)),
           pl.BlockSpec((D, D), lambda i: (0, 0))],
        out_specs=[pl.BlockSpec((bn, D), lambda i: (i, 0))]
        + [pl.BlockSpec((bn, CW), lambda i: (i, 0))] * 4,
        out_shape=[jax.ShapeDtypeStruct((n, D), jnp.float32)]
        + [jax.ShapeDtypeStruct((n, CW), jnp.float32)] * 4,
    )(*aggs, cnt, x, wc, bc, wl2)


def _tc_layer2(aggs, cnt, h, we, be, bn=1000):
    n = h.shape[0]

    def body(a0, a1, a2, a3, c_ref, h_ref, we_ref, be_ref, o_ref):
        agg = jnp.concatenate(
            [a0[...], a1[...], a2[...], a3[...]], axis=1)
        recip = 1.0 / jnp.clip(c_ref[:, 0:1], 1.0, None)
        o_ref[...] = (agg * recip
                      + jnp.dot(h_ref[...], we_ref[...],
                                preferred_element_type=jnp.float32)
                      + be_ref[...])

    return pl.pallas_call(
        body,
        grid=(n // bn,),
        in_specs=[pl.BlockSpec((bn, CW), lambda i: (i, 0))] * 4
        + [pl.BlockSpec((bn, CW), lambda i: (i, 0)),
           pl.BlockSpec((bn, D), lambda i: (i, 0)),
           pl.BlockSpec((D, D), lambda i: (0, 0)),
           pl.BlockSpec((1, D), lambda i: (0, 0))],
        out_specs=pl.BlockSpec((bn, D), lambda i: (i, 0)),
        out_shape=jax.ShapeDtypeStruct((n, D), jnp.float32),
    )(*aggs, cnt, h, we, be)


def _prep_edges(ei, trash_row):
    e = ei.shape[1]
    nb = -(-e // (NTILES * B))
    ep = NTILES * nb * B
    src = ei[0].astype(jnp.int32)
    dst = ei[1].astype(jnp.int32)
    if ep > e:
        pad = ep - e
        src = jnp.concatenate([src, jnp.zeros((pad,), jnp.int32)])
        dst = jnp.concatenate(
            [dst, jnp.full((pad,), trash_row, jnp.int32)])
    return src.reshape(NTILES, nb, B), dst.reshape(NTILES, nb, B)


def kernel(x_user, x_item, edge_index_ui, edge_index_iu,
           conv1_ui_Wl, conv1_ui_bl, conv1_ui_Wr,
           conv1_iu_Wl, conv1_iu_bl, conv1_iu_Wr,
           lin1_user_W, lin1_user_b,
           lin1_item_W, lin1_item_b,
           conv2_ui_Wl, conv2_ui_bl, conv2_ui_Wr,
           conv2_iu_Wl, conv2_iu_bl, conv2_iu_Wr,
           lin2_user_W, lin2_user_b,
           lin2_item_W, lin2_item_b):
    n_user = x_user.shape[0]
    n_item = x_item.shape[0]
    assert n_user == n_item
    n = n_user
    half = -(-n // (2 * NTILES * B)) * (NTILES * B)

    wc_u = conv1_iu_Wr + lin1_user_W
    bc_u = (conv1_iu_bl + lin1_user_b).reshape(1, D)
    wc_i = conv1_ui_Wr + lin1_item_W
    bc_i = (conv1_ui_bl + lin1_item_b).reshape(1, D)
    we_u = conv2_iu_Wr + lin2_user_W
    be_u = (conv2_iu_bl + lin2_user_b).reshape(1, D)
    we_i = conv2_ui_Wr + lin2_item_W
    be_i = (conv2_ui_bl + lin2_item_b).reshape(1, D)

    accr = -(-(n + 1) // (NTILES * B)) * (NTILES * B)

    src3_ui, dst3_ui = _prep_edges(edge_index_ui, n)
    src3_iu, dst3_iu = _prep_edges(edge_index_iu, n)
    src3 = jnp.stack([src3_ui, src3_iu])
    dst3 = jnp.stack([dst3_ui, dst3_iu])

    xcat = jnp.concatenate([x_user, x_item])
    y1 = _tc_stacked_matmul(xcat, conv1_ui_Wl, conv1_iu_Wl)
    cnt_ui, cnt_iu = _sc_counts(dst3_ui, dst3_iu, half)
    o = _sc_seg_sum_dual(y1, src3, dst3, accr)
    agg_ui1, agg_iu1 = o[:4], o[4:]

    h1_user, *y2u = _tc_layer1(agg_iu1, cnt_iu, x_user, wc_u, bc_u,
                               conv2_ui_Wl)
    h1_item, *y2i = _tc_layer1(agg_ui1, cnt_ui, x_item, wc_i, bc_i,
                               conv2_iu_Wl)

    y2 = [jnp.stack([u, i]) for u, i in zip(y2u, y2i)]
    o2 = _sc_seg_sum_dual(y2, src3, dst3, accr)
    agg_ui2, agg_iu2 = o2[:4], o2[4:]

    out_user = _tc_layer2(agg_iu2, cnt_iu, h1_user, we_u, be_u)
    out_item = _tc_layer2(agg_ui2, cnt_ui, h1_item, we_i, be_i)
    return (out_user, out_item)
